# baseline (speedup 1.0000x reference)
import functools

import jax
import jax.numpy as jnp
from jax import lax
from jax.experimental import pallas as pl
from jax.experimental.pallas import tpu as pltpu
from jax.experimental.pallas import tpu_sc as plsc

WORD_NUM = 1000000
EMBED = 64
HIDDEN = 64
LABELS = 128
BATCH = 4096
SEQ = 200
HALF = SEQ // 2

_INFO = plsc.get_sparse_core_info()
NC = _INFO.num_cores
NS = _INFO.num_subcores
NW = NC * NS
SPT = BATCH // NW
NLANE = EMBED // 16


def _pool_body(x2_hbm, emb_hbm, out_hbm, idx_v, rows0, rows1, acc_v, sem):
    wid = lax.axis_index("s") * NC + lax.axis_index("c")
    pltpu.sync_copy(x2_hbm.at[pl.ds(wid * 2 * SPT, 2 * SPT)], idx_v)

    zero = jnp.zeros((16,), jnp.float32)

    def seq_body(s, carry):
        cp0 = pltpu.async_copy(emb_hbm.at[idx_v.at[2 * s]], rows0, sem)
        cp1 = pltpu.async_copy(emb_hbm.at[idx_v.at[2 * s + 1]], rows1, sem)
        cp0.wait()
        cp1.wait()

        def red(r, accs):
            return tuple(
                accs[j]
                + rows0[r, pl.ds(16 * j, 16)]
                + rows1[r, pl.ds(16 * j, 16)]
                for j in range(NLANE)
            )

        accs = lax.fori_loop(0, HALF, red, (zero,) * NLANE)
        for j in range(NLANE):
            acc_v[pl.ds(s * EMBED + 16 * j, 16)] = accs[j]
        return carry

    lax.fori_loop(0, SPT, seq_body, 0)
    pltpu.sync_copy(acc_v, out_hbm.at[pl.ds(wid * SPT * EMBED, SPT * EMBED)])


@functools.partial(jax.jit, static_argnames=())
def _pool(x2, emb):
    mesh = plsc.VectorSubcoreMesh(core_axis_name="c", subcore_axis_name="s")
    return pl.kernel(
        _pool_body,
        out_type=jax.ShapeDtypeStruct((BATCH * EMBED,), jnp.float32),
        mesh=mesh,
        compiler_params=pltpu.CompilerParams(use_tc_tiling_on_sc=False),
        scratch_types=[
            pltpu.VMEM((2 * SPT, HALF), jnp.int32),
            pltpu.VMEM((HALF, EMBED), jnp.float32),
            pltpu.VMEM((HALF, EMBED), jnp.float32),
            pltpu.VMEM((SPT * EMBED,), jnp.float32),
            pltpu.SemaphoreType.DMA,
        ],
    )(x2, emb)


def _mlp_body(p_ref, wh_ref, bh_ref, wo_ref, bo_ref, o_ref):
    p = p_ref[...] * (1.0 / SEQ)
    h = jnp.dot(p, wh_ref[...], preferred_element_type=jnp.float32) + bh_ref[...]
    o_ref[...] = (
        jnp.dot(h, wo_ref[...], preferred_element_type=jnp.float32) + bo_ref[...]
    )


def _mlp(pooled, W_h, b_h, W_o, b_o):
    bb = 1024
    return pl.pallas_call(
        _mlp_body,
        grid=(BATCH // bb,),
        in_specs=[
            pl.BlockSpec((bb, EMBED), lambda i: (i, 0)),
            pl.BlockSpec((EMBED, HIDDEN), lambda i: (0, 0)),
            pl.BlockSpec((1, HIDDEN), lambda i: (0, 0)),
            pl.BlockSpec((HIDDEN, LABELS), lambda i: (0, 0)),
            pl.BlockSpec((1, LABELS), lambda i: (0, 0)),
        ],
        out_specs=pl.BlockSpec((bb, LABELS), lambda i: (i, 0)),
        out_shape=jax.ShapeDtypeStruct((BATCH, LABELS), jnp.float32),
    )(pooled, W_h, b_h.reshape(1, HIDDEN), W_o, b_o.reshape(1, LABELS))


def kernel(X, emb, W_h, b_h, W_o, b_o):
    x2 = X.astype(jnp.int32).reshape(2 * BATCH, HALF)
    pooled = _pool(x2, emb).reshape(BATCH, EMBED)
    return _mlp(pooled, W_h, b_h, W_o, b_o)


# speedup vs baseline: 1.1186x; 1.1186x over previous
import functools

import jax
import jax.numpy as jnp
from jax import lax
from jax.experimental import pallas as pl
from jax.experimental.pallas import tpu as pltpu
from jax.experimental.pallas import tpu_sc as plsc

WORD_NUM = 1000000
EMBED = 64
HIDDEN = 64
LABELS = 128
BATCH = 4096
SEQ = 200
HALF = SEQ // 2

_INFO = plsc.get_sparse_core_info()
NC = _INFO.num_cores
NS = _INFO.num_subcores
NW = NC * NS
SPT = BATCH // NW
NLANE = EMBED // 16


CH0 = 104
CH1 = 96


def _pool_body(x_hbm, emb_hbm, out_hbm, idx_v, rows_v, acc_v, sem0, sem1):
    wid = lax.axis_index("s") * NC + lax.axis_index("c")
    pltpu.sync_copy(x_hbm.at[pl.ds(wid * SPT, SPT)], idx_v)

    sems = (sem0, sem1)

    def issue(s, b):
        pltpu.async_copy(
            emb_hbm.at[idx_v.at[s, pl.ds(0, CH0)]],
            rows_v.at[b, pl.ds(0, CH0)],
            sems[b],
        )
        pltpu.async_copy(
            emb_hbm.at[idx_v.at[s, pl.ds(CH0, CH1)]],
            rows_v.at[b, pl.ds(CH0, CH1)],
            sems[b],
        )

    def drain(b):
        pltpu.make_async_copy(
            emb_hbm.at[idx_v.at[0, pl.ds(0, CH0)]],
            rows_v.at[b, pl.ds(0, CH0)],
            sems[b],
        ).wait()
        pltpu.make_async_copy(
            emb_hbm.at[idx_v.at[0, pl.ds(CH0, CH1)]],
            rows_v.at[b, pl.ds(CH0, CH1)],
            sems[b],
        ).wait()

    zero = jnp.zeros((16,), jnp.float32)

    def reduce_store(s, b):
        def red(r, accs):
            return tuple(
                accs[j] + rows_v[b, r, pl.ds(16 * j, 16)] for j in range(NLANE)
            )

        accs = lax.fori_loop(0, SEQ, red, (zero,) * NLANE)
        for j in range(NLANE):
            acc_v[pl.ds(s * EMBED + 16 * j, 16)] = accs[j]

    issue(0, 0)
    issue(1, 1)

    def pair_body(p, carry):
        s0 = 2 * p
        for b in range(2):
            s = s0 + b
            drain(b)
            reduce_store(s, b)

            @pl.when(s + 2 < SPT)
            def _():
                issue(s + 2, b)

        return carry

    lax.fori_loop(0, SPT // 2, pair_body, 0)
    pltpu.sync_copy(acc_v, out_hbm.at[pl.ds(wid * SPT * EMBED, SPT * EMBED)])


@functools.partial(jax.jit, static_argnames=())
def _pool(x, emb):
    mesh = plsc.VectorSubcoreMesh(core_axis_name="c", subcore_axis_name="s")
    return pl.kernel(
        _pool_body,
        out_type=jax.ShapeDtypeStruct((BATCH * EMBED,), jnp.float32),
        mesh=mesh,
        compiler_params=pltpu.CompilerParams(use_tc_tiling_on_sc=False),
        scratch_types=[
            pltpu.VMEM((SPT, SEQ), jnp.int32),
            pltpu.VMEM((2, SEQ, EMBED), jnp.float32),
            pltpu.VMEM((SPT * EMBED,), jnp.float32),
            pltpu.SemaphoreType.DMA,
            pltpu.SemaphoreType.DMA,
        ],
    )(x, emb)


def _mlp_body(p_ref, wh_ref, bh_ref, wo_ref, bo_ref, o_ref):
    p = p_ref[...] * (1.0 / SEQ)
    h = jnp.dot(p, wh_ref[...], preferred_element_type=jnp.float32) + bh_ref[...]
    o_ref[...] = (
        jnp.dot(h, wo_ref[...], preferred_element_type=jnp.float32) + bo_ref[...]
    )


def _mlp(pooled, W_h, b_h, W_o, b_o):
    bb = 1024
    return pl.pallas_call(
        _mlp_body,
        grid=(BATCH // bb,),
        in_specs=[
            pl.BlockSpec((bb, EMBED), lambda i: (i, 0)),
            pl.BlockSpec((EMBED, HIDDEN), lambda i: (0, 0)),
            pl.BlockSpec((1, HIDDEN), lambda i: (0, 0)),
            pl.BlockSpec((HIDDEN, LABELS), lambda i: (0, 0)),
            pl.BlockSpec((1, LABELS), lambda i: (0, 0)),
        ],
        out_specs=pl.BlockSpec((bb, LABELS), lambda i: (i, 0)),
        out_shape=jax.ShapeDtypeStruct((BATCH, LABELS), jnp.float32),
    )(pooled, W_h, b_h.reshape(1, HIDDEN), W_o, b_o.reshape(1, LABELS))


def kernel(X, emb, W_h, b_h, W_o, b_o):
    pooled = _pool(X.astype(jnp.int32), emb).reshape(BATCH, EMBED)
    return _mlp(pooled, W_h, b_h, W_o, b_o)
